# baseline (speedup 1.0000x reference)
import jax
import jax.numpy as jnp
from jax import lax
from jax.experimental import pallas as pl
from jax.experimental.pallas import tpu as pltpu
from jax.experimental.pallas import tpu_sc as plsc

B, T, A, V, K, L = 64, 32, 4, 2048, 64, 16
NEG = float("-inf")
IMAX = 2147483647
CAP = 4096


def _iota():
    return lax.iota(jnp.int32, L)


def _spl_i(x):
    return jnp.broadcast_to(jnp.int32(x), (L,))


def _spl_f(x):
    return jnp.broadcast_to(jnp.float32(x), (L,))


def _dg(x, idx):
    return x.at[idx].get(mode="promise_in_bounds")


def _lexgt(v1, f1, v2, f2):
    return (v1 > v2) | ((v1 == v2) & (f1 < f2))


def _insert4(R, x):
    R0, R1, R2, R3 = R
    m0, m1, m2, m3 = x > R0, x > R1, x > R2, x > R3
    n0 = jnp.where(m0, x, R0)
    n1 = jnp.where(m0, R0, jnp.where(m1, x, R1))
    n2 = jnp.where(m1, R1, jnp.where(m2, x, R2))
    n3 = jnp.where(m2, R2, jnp.where(m3, x, R3))
    return (n0, n1, n2, n3)


def _extract_topk(cv, cf, nvregs, kk, wv_ref, wf_ref):
    iota = _iota()

    def kbody(k, _):
        def scan(kv, car):
            v, f, s = car
            base = jnp.broadcast_to(kv * L, (L,)) + iota
            xv = plsc.load_gather(cv, [base])
            xf = plsc.load_gather(cf, [base])
            take = _lexgt(xv, xf, v, f)
            return (jnp.where(take, xv, v), jnp.where(take, xf, f),
                    jnp.where(take, base, s))

        v, f, s = lax.fori_loop(
            0, nvregs, scan, (_spl_f(NEG), _spl_i(IMAX), _spl_i(0)))
        for d in (8, 4, 2, 1):
            p = iota ^ d
            pv, pf, ps = _dg(v, p), _dg(f, p), _dg(s, p)
            take = _lexgt(pv, pf, v, f)
            v = jnp.where(take, pv, v)
            f = jnp.where(take, pf, f)
            s = jnp.where(take, ps, s)
        lane0 = iota == 0
        kidx = jnp.broadcast_to(k, (L,))
        plsc.store_scatter(wv_ref, [kidx], v, mask=lane0)
        plsc.store_scatter(wf_ref, [kidx], f, mask=lane0)
        plsc.store_scatter(cv, [s], _spl_f(NEG), mask=lane0)
        return 0

    lax.fori_loop(0, kk, kbody, 0)


def _body(tl_hbm, arg_hbm, ids_hbm, sc_hbm,
          rows_v, gmax_v, bounds_v, s_v, g_addr, g_flat, g_sc,
          c_val, c_flat, win_val, win_flat,
          score_st, t_st, ids_st, tmp_st, tl_v, outb, sem):
    iota = _iota()
    cid = lax.axis_index("c")
    sid = lax.axis_index("s")
    wid = sid * 2 + cid

    def batch_body(bi, _):
        b = wid * 2 + bi

        pltpu.sync_copy(tl_hbm.at[pl.ds(b * T, T)], tl_v.at[pl.ds(0, T)])
        for kv in range(2):
            c_val[pl.ds(kv * L, L)] = tl_v[pl.ds(kv * L, L)]
            c_flat[pl.ds(kv * L, L)] = iota + kv * L
        c_val[pl.ds(2 * L, L)] = _spl_f(NEG)
        c_flat[pl.ds(2 * L, L)] = _spl_i(IMAX)
        _extract_topk(c_val, c_flat, 2, T, win_val, win_flat)
        for kv in range(2):
            w = win_flat[pl.ds(kv * L, L)]
            score_st[pl.ds(kv * L, L)] = win_val[pl.ds(kv * L, L)]
            t_st[pl.ds(kv * L, L)] = w
            ids_st[pl.ds(kv * L, L)] = w
        for kv in range(2, 4):
            score_st[pl.ds(kv * L, L)] = _spl_f(NEG)
            t_st[pl.ds(kv * L, L)] = _spl_i(0)
            ids_st[pl.ds(kv * L, L)] = _spl_i(0)

        def fire_rows(ap):
            def fire(t, _):
                base = ((b * T + t) * A + ap) * V
                pltpu.async_copy(arg_hbm.at[pl.ds(base, V)],
                                 rows_v.at[pl.ds(t * V, V)], sem)
                return 0
            lax.fori_loop(0, T, fire, 0)

        fire_rows(0)

        def ap_body(ap, _):
            pltpu.make_async_copy(arg_hbm.at[pl.ds(0, T * V)], rows_v,
                                  sem).wait()

            def gmax_row(t, _):
                tv = t * V
                accs = [_spl_f(NEG) for _ in range(4)]
                for j in range(32):
                    for s in range(4):
                        x = rows_v[pl.ds(tv + (s * L + 64 * j), L)]
                        accs[s] = jnp.maximum(accs[s], x)
                gb = jnp.broadcast_to(t * 64, (L,)) + iota
                for s in range(4):
                    plsc.store_scatter(gmax_v, [gb + s * L], accs[s])
                return 0
            lax.fori_loop(0, T, gmax_row, 0)

            def bv_body(i, R):
                t_spl = plsc.load_gather(t_st, [jnp.broadcast_to(i, (L,))])
                s_spl = plsc.load_gather(score_st,
                                         [jnp.broadcast_to(i, (L,))])
                gb = t_spl * 64 + iota
                bb = jnp.broadcast_to(i * 64, (L,)) + iota
                bv = _spl_f(NEG)
                for s in range(4):
                    bnd = plsc.load_gather(gmax_v, [gb + s * L]) + s_spl
                    plsc.store_scatter(bounds_v, [bb + s * L], bnd)
                    bv = jnp.maximum(bv, bnd)
                return _insert4(R, bv)
            R = lax.fori_loop(0, K, bv_body,
                              (_spl_f(NEG),) * 4)
            tau0 = jnp.min(R[3])

            lane15 = _spl_i(15)

            def sc_scan(kv, off):
                base = jnp.broadcast_to(kv * L, (L,)) + iota
                x = plsc.load_gather(bounds_v, [base])
                msk = x >= tau0
                pos = off + plsc.cumsum(msk.astype(jnp.int32)) - 1
                plsc.store_scatter(s_v, [pos], x, mask=msk)
                return _dg(pos, lane15) + 1
            n_s_vec = lax.fori_loop(0, K * 4, sc_scan, _spl_i(0))
            plsc.store_scatter(s_v, [n_s_vec + iota], _spl_f(NEG))
            n_s = jnp.max(n_s_vec)

            def s4_body(kv, R):
                base = jnp.broadcast_to(kv * L, (L,)) + iota
                return _insert4(R, plsc.load_gather(s_v, [base]))
            R = lax.fori_loop(0, (n_s + L - 1) // L, s4_body,
                              (_spl_f(NEG),) * 4)
            tau1 = jnp.min(R[3])

            def en_body(i, ng):
                t_spl = plsc.load_gather(t_st, [jnp.broadcast_to(i, (L,))])
                s_spl = plsc.load_gather(score_st,
                                         [jnp.broadcast_to(i, (L,))])
                for s in range(4):
                    x = plsc.load_gather(
                        bounds_v,
                        [jnp.broadcast_to(i * 64 + s * L, (L,)) + iota])
                    msk = x >= tau1
                    pos = ng + plsc.cumsum(msk.astype(jnp.int32)) - 1
                    addr = t_spl * V + iota + s * L
                    flat = jnp.broadcast_to(i * V + s * L, (L,)) + iota
                    plsc.store_scatter(g_addr, [pos], addr, mask=msk)
                    plsc.store_scatter(g_flat, [pos], flat, mask=msk)
                    plsc.store_scatter(g_sc, [pos], s_spl, mask=msk)
                    ng = _dg(pos, lane15) + 1
                return ng
            ng_vec = lax.fori_loop(0, K, en_body, _spl_i(0))
            plsc.store_scatter(g_addr, [ng_vec + iota], _spl_i(0))
            plsc.store_scatter(g_flat, [ng_vec + iota], _spl_i(0))
            plsc.store_scatter(g_sc, [ng_vec + iota], _spl_f(NEG))
            ng = jnp.max(ng_vec)

            def cd_body(gv, nc):
                base = jnp.broadcast_to(gv * L, (L,)) + iota
                ab = plsc.load_gather(g_addr, [base])
                fb = plsc.load_gather(g_flat, [base])
                sc = plsc.load_gather(g_sc, [base])
                for j in range(32):
                    val = plsc.load_gather(rows_v, [ab + 64 * j]) + sc
                    msk = val >= tau1
                    pos = nc + plsc.cumsum(msk.astype(jnp.int32)) - 1
                    msk2 = msk & (pos < CAP)
                    plsc.store_scatter(c_val, [pos], val, mask=msk2)
                    plsc.store_scatter(c_flat, [pos], fb + 64 * j, mask=msk2)
                    nc = _dg(pos, lane15) + 1
                return nc
            nc_vec = lax.fori_loop(0, (ng + L - 1) // L, cd_body, _spl_i(0))
            nc_vec = jnp.minimum(nc_vec, jnp.int32(CAP))
            plsc.store_scatter(c_val, [nc_vec + iota], _spl_f(NEG))
            plsc.store_scatter(c_flat, [nc_vec + iota], _spl_i(IMAX))
            nc = jnp.max(nc_vec)

            @pl.when(ap < A - 1)
            def _():
                fire_rows(ap + 1)

            _extract_topk(c_val, c_flat, (nc + L - 1) // L, K,
                          win_val, win_flat)

            for c in range(5):
                for kv in range(4):
                    wf = win_flat[pl.ds(kv * L, L)]
                    beams = wf >> 11
                    g = plsc.load_gather(
                        ids_st, [jnp.broadcast_to(c * K, (L,)) + beams])
                    tmp_st[pl.ds(kv * L, L)] = g
                for kv in range(4):
                    ids_st[pl.ds(c * K + kv * L, L)] = tmp_st[pl.ds(kv * L, L)]
            for kv in range(4):
                wf = win_flat[pl.ds(kv * L, L)]
                score_st[pl.ds(kv * L, L)] = win_val[pl.ds(kv * L, L)]
                t_st[pl.ds(kv * L, L)] = ids_st[pl.ds(kv * L, L)]
                plsc.store_scatter(
                    ids_st,
                    [jnp.broadcast_to((ap + 1) * K + kv * L, (L,)) + iota],
                    wf & (V - 1))
            return 0

        lax.fori_loop(0, A, ap_body, 0)

        for kv in range(4):
            for c in range(5):
                plsc.store_scatter(
                    outb, [(iota + kv * L) * 5 + c],
                    ids_st[pl.ds(c * K + kv * L, L)])
        pltpu.sync_copy(outb.at[pl.ds(0, 320)], ids_hbm.at[pl.ds(b * 320, 320)])
        pltpu.sync_copy(score_st.at[pl.ds(0, K)], sc_hbm.at[pl.ds(b * K, K)])
        return 0

    lax.fori_loop(0, 2, batch_body, 0)


def kernel(tactic_logits, arg_logits):
    tl_flat = tactic_logits.reshape(-1)
    arg_flat = arg_logits.reshape(-1)
    mesh = plsc.VectorSubcoreMesh(core_axis_name="c", subcore_axis_name="s",
                                  num_cores=2, num_subcores=16)
    f = pl.kernel(
        _body,
        out_type=(
            jax.ShapeDtypeStruct((B * 320,), jnp.int32),
            jax.ShapeDtypeStruct((B * K,), jnp.float32),
        ),
        mesh=mesh,
        compiler_params=pltpu.CompilerParams(needs_layout_passes=False),
        scratch_types=[
            pltpu.VMEM((T * V,), jnp.float32),
            pltpu.VMEM((T * 64,), jnp.float32),
            pltpu.VMEM((K * 64,), jnp.float32),
            pltpu.VMEM((CAP + 128,), jnp.float32),
            pltpu.VMEM((CAP + 128,), jnp.int32),
            pltpu.VMEM((CAP + 128,), jnp.int32),
            pltpu.VMEM((CAP + 128,), jnp.float32),
            pltpu.VMEM((CAP + 128,), jnp.float32),
            pltpu.VMEM((CAP + 128,), jnp.int32),
            pltpu.VMEM((128,), jnp.float32),
            pltpu.VMEM((128,), jnp.int32),
            pltpu.VMEM((128,), jnp.float32),
            pltpu.VMEM((128,), jnp.int32),
            pltpu.VMEM((5 * 128,), jnp.int32),
            pltpu.VMEM((128,), jnp.int32),
            pltpu.VMEM((128,), jnp.float32),
            pltpu.VMEM((384,), jnp.int32),
            pltpu.SemaphoreType.DMA,
        ],
    )
    ids_f, sc_f = f(tl_flat, arg_flat)
    return ids_f.reshape(B, K, 5), sc_f.reshape(B, K)


# speedup vs baseline: 1.1527x; 1.1527x over previous
import jax
import jax.numpy as jnp
from jax import lax
from jax.experimental import pallas as pl
from jax.experimental.pallas import tpu as pltpu
from jax.experimental.pallas import tpu_sc as plsc

B, T, A, V, K, L = 64, 32, 4, 2048, 64, 16
NEG = float("-inf")
IMAX = 2147483647
CAP = 4096


def _iota():
    return lax.iota(jnp.int32, L)


def _spl_i(x):
    return jnp.broadcast_to(jnp.int32(x), (L,))


def _spl_f(x):
    return jnp.broadcast_to(jnp.float32(x), (L,))


def _dg(x, idx):
    return x.at[idx].get(mode="promise_in_bounds")


def _lexgt(v1, f1, v2, f2):
    return (v1 > v2) | ((v1 == v2) & (f1 < f2))


def _insert4(R, x):
    R0, R1, R2, R3 = R
    m0, m1, m2, m3 = x > R0, x > R1, x > R2, x > R3
    n0 = jnp.where(m0, x, R0)
    n1 = jnp.where(m0, R0, jnp.where(m1, x, R1))
    n2 = jnp.where(m1, R1, jnp.where(m2, x, R2))
    n3 = jnp.where(m2, R2, jnp.where(m3, x, R3))
    return (n0, n1, n2, n3)


def _ce_intra(v, f, j, k, iota):
    isfirst = (iota & j) == 0
    dirv = (iota & k) == 0 if k < L else None
    nv, nf = [], []
    for r in range(4):
        pv, pf = _dg(v[r], iota ^ j), _dg(f[r], iota ^ j)
        takep = _lexgt(pv, pf, v[r], f[r])
        bigv = jnp.where(takep, pv, v[r])
        bigf = jnp.where(takep, pf, f[r])
        smv = jnp.where(takep, v[r], pv)
        smf = jnp.where(takep, f[r], pf)
        if dirv is None:
            want = isfirst if ((r * L) & k) == 0 else ~isfirst
        else:
            want = isfirst == dirv
        nv.append(jnp.where(want, bigv, smv))
        nf.append(jnp.where(want, bigf, smf))
    return nv, nf


def _ce_inter(v, f, j, k):
    step = j // L
    nv, nf = list(v), list(f)
    for ra in range(4):
        rb = ra + step
        if ra & step or rb > 3:
            continue
        takeb = _lexgt(v[rb], f[rb], v[ra], f[ra])
        bigv = jnp.where(takeb, v[rb], v[ra])
        bigf = jnp.where(takeb, f[rb], f[ra])
        smv = jnp.where(takeb, v[ra], v[rb])
        smf = jnp.where(takeb, f[ra], f[rb])
        if ((ra * L) & k) == 0:
            nv[ra], nf[ra], nv[rb], nf[rb] = bigv, bigf, smv, smf
        else:
            nv[ra], nf[ra], nv[rb], nf[rb] = smv, smf, bigv, bigf
    return nv, nf


def _sort64_desc(v, f, iota):
    for k in (2, 4, 8, 16, 32, 64):
        j = k // 2
        while j >= 1:
            if j >= L:
                v, f = _ce_inter(v, f, j, k)
            else:
                v, f = _ce_intra(v, f, j, k, iota)
            j //= 2
    return v, f


def _merge_desc(Rv, Rf, Bv, Bf, iota):
    rev = L - 1 - iota
    v, f = [], []
    for r in range(4):
        brv, brf = _dg(Bv[3 - r], rev), _dg(Bf[3 - r], rev)
        take = _lexgt(brv, brf, Rv[r], Rf[r])
        v.append(jnp.where(take, brv, Rv[r]))
        f.append(jnp.where(take, brf, Rf[r]))
    for j in (32, 16, 8, 4, 2, 1):
        if j >= L:
            v, f = _ce_inter(v, f, j, 64)
        else:
            v, f = _ce_intra(v, f, j, 64, iota)
    return v, f


def _bitonic_topk(cv, cf, nb, wv_ref, wf_ref):
    iota = _iota()

    def blk_body(blk, car):
        Rv, Rf = list(car[0:4]), list(car[4:8])
        base = blk * 64
        Bv = [cv[pl.ds(base + r * L, L)] for r in range(4)]
        Bf = [cf[pl.ds(base + r * L, L)] for r in range(4)]
        Bv, Bf = _sort64_desc(Bv, Bf, iota)
        Rv, Rf = _merge_desc(Rv, Rf, Bv, Bf, iota)
        return (*Rv, *Rf)

    car = lax.fori_loop(
        0, nb, blk_body,
        tuple([_spl_f(NEG)] * 4 + [_spl_i(IMAX)] * 4))
    for r in range(4):
        wv_ref[pl.ds(r * L, L)] = car[r]
        wf_ref[pl.ds(r * L, L)] = car[4 + r]


def _body(tl_hbm, arg_hbm, ids_hbm, sc_hbm,
          rows_v, gmax_v, bounds_v, s_v, g_addr, g_flat, g_sc,
          c_val, c_flat, win_val, win_flat,
          score_st, t_st, ids_st, tmp_st, tl_v, outb, aux, sem):
    iota = _iota()
    cid = lax.axis_index("c")
    sid = lax.axis_index("s")
    wid = sid * 2 + cid

    def batch_body(bi, _):
        b = wid * 2 + bi

        pltpu.sync_copy(tl_hbm.at[pl.ds(b * T, T)], tl_v.at[pl.ds(0, T)])
        for kv in range(2):
            c_val[pl.ds(kv * L, L)] = tl_v[pl.ds(kv * L, L)]
            c_flat[pl.ds(kv * L, L)] = iota + kv * L
        for kv in range(2, 4):
            c_val[pl.ds(kv * L, L)] = _spl_f(NEG)
            c_flat[pl.ds(kv * L, L)] = _spl_i(IMAX)

        for kv in range(2):
            aux[pl.ds(32 + kv * L, L)] = iota + kv * L
            aux[pl.ds(64 + kv * L, L)] = iota + kv * L

        def fire_rows(ap, nt):
            def fire(i, _):
                tid = jnp.max(plsc.load_gather(
                    aux, [jnp.broadcast_to(32 + i, (L,))]))
                basep = ((b * T + tid) * A + ap) * V
                pltpu.async_copy(arg_hbm.at[pl.ds(basep, V)],
                                 rows_v.at[pl.ds(i * V, V)], sem)
                return 0
            lax.fori_loop(0, nt, fire, 0)

        fire_rows(0, jnp.int32(T))
        _bitonic_topk(c_val, c_flat, 1, win_val, win_flat)
        for kv in range(2):
            w = win_flat[pl.ds(kv * L, L)]
            score_st[pl.ds(kv * L, L)] = win_val[pl.ds(kv * L, L)]
            t_st[pl.ds(kv * L, L)] = w
            ids_st[pl.ds(kv * L, L)] = w
        for kv in range(2, 4):
            score_st[pl.ds(kv * L, L)] = _spl_f(NEG)
            t_st[pl.ds(kv * L, L)] = _spl_i(0)
            ids_st[pl.ds(kv * L, L)] = _spl_i(0)

        def ap_body(ap, nt_prev):
            def dr(i, _):
                pltpu.make_async_copy(arg_hbm.at[pl.ds(0, V)],
                                      rows_v.at[pl.ds(0, V)], sem).wait()
                return 0
            lax.fori_loop(0, nt_prev, dr, 0)

            def gmax_row(t, _):
                tv = t * V
                accs = [_spl_f(NEG) for _ in range(4)]
                for j in range(32):
                    for s in range(4):
                        x = rows_v[pl.ds(tv + (s * L + 64 * j), L)]
                        accs[s] = jnp.maximum(accs[s], x)
                for s in range(4):
                    gmax_v[pl.ds(t * 64 + s * L, L)] = accs[s]
                return 0
            lax.fori_loop(0, nt_prev, gmax_row, 0)

            def bv_body(i, R):
                t_spl = plsc.load_gather(t_st, [jnp.broadcast_to(i, (L,))])
                s_spl = plsc.load_gather(score_st,
                                         [jnp.broadcast_to(i, (L,))])
                sl_spl = plsc.load_gather(aux, [t_spl + 64])
                gb = sl_spl * 64 + iota
                bv = _spl_f(NEG)
                for s in range(4):
                    bnd = plsc.load_gather(gmax_v, [gb + s * L]) + s_spl
                    bounds_v[pl.ds(i * 64 + s * L, L)] = bnd
                    bv = jnp.maximum(bv, bnd)
                return _insert4(R, bv)
            R = lax.fori_loop(0, K, bv_body,
                              (_spl_f(NEG),) * 4, unroll=2)
            tau0 = jnp.min(R[3])

            lane15 = _spl_i(15)

            def sc_scan(kv, off):
                x = bounds_v[pl.ds(kv * L, L)]
                msk = x >= tau0
                pos = off + plsc.cumsum(msk.astype(jnp.int32)) - 1
                plsc.store_scatter(s_v, [pos], x, mask=msk)
                return _dg(pos, lane15) + 1
            n_s_vec = lax.fori_loop(0, K * 4, sc_scan, _spl_i(0), unroll=4)
            plsc.store_scatter(s_v, [n_s_vec + iota], _spl_f(NEG))
            n_s = jnp.max(n_s_vec)

            def s4_body(kv, R):
                return _insert4(R, s_v[pl.ds(kv * L, L)])
            R = lax.fori_loop(0, (n_s + L - 1) // L, s4_body,
                              (_spl_f(NEG),) * 4)
            tau1 = jnp.min(R[3])

            def en_body(i, ng):
                t_spl = plsc.load_gather(t_st, [jnp.broadcast_to(i, (L,))])
                s_spl = plsc.load_gather(score_st,
                                         [jnp.broadcast_to(i, (L,))])
                sl_spl = plsc.load_gather(aux, [t_spl + 64])
                for s in range(4):
                    x = bounds_v[pl.ds(i * 64 + s * L, L)]
                    msk = x >= tau1
                    pos = ng + plsc.cumsum(msk.astype(jnp.int32)) - 1
                    addr = sl_spl * V + iota + s * L
                    flat = jnp.broadcast_to(i * V + s * L, (L,)) + iota
                    plsc.store_scatter(g_addr, [pos], addr, mask=msk)
                    plsc.store_scatter(g_flat, [pos], flat, mask=msk)
                    plsc.store_scatter(g_sc, [pos], s_spl, mask=msk)
                    ng = _dg(pos, lane15) + 1
                return ng
            ng_vec = lax.fori_loop(0, K, en_body, _spl_i(0), unroll=2)
            plsc.store_scatter(g_addr, [ng_vec + iota], _spl_i(0))
            plsc.store_scatter(g_flat, [ng_vec + iota], _spl_i(0))
            plsc.store_scatter(g_sc, [ng_vec + iota], _spl_f(NEG))
            ng = jnp.max(ng_vec)

            def cd_body(gv, nc):
                ab = g_addr[pl.ds(gv * L, L)]
                fb = g_flat[pl.ds(gv * L, L)]
                sc = g_sc[pl.ds(gv * L, L)]
                for j in range(32):
                    val = plsc.load_gather(rows_v, [ab + 64 * j]) + sc
                    msk = val >= tau1
                    pos = nc + plsc.cumsum(msk.astype(jnp.int32)) - 1
                    msk2 = msk & (pos < CAP)
                    plsc.store_scatter(c_val, [pos], val, mask=msk2)
                    plsc.store_scatter(c_flat, [pos], fb + 64 * j, mask=msk2)
                    nc = _dg(pos, lane15) + 1
                return nc
            nc_vec = lax.fori_loop(0, (ng + L - 1) // L, cd_body, _spl_i(0))
            nc_vec = jnp.minimum(nc_vec, jnp.int32(CAP))
            plsc.store_scatter(c_val, [nc_vec + iota], _spl_f(NEG))
            plsc.store_scatter(c_flat, [nc_vec + iota], _spl_i(IMAX))
            nc = jnp.max(nc_vec)

            ncv = (nc + L - 1) // L

            def c4_body(kv, R):
                return _insert4(R, c_val[pl.ds(kv * L, L)])
            R = lax.fori_loop(0, ncv, c4_body, (_spl_f(NEG),) * 4)
            tau2 = jnp.min(R[3])

            def cc_body(kv, off):
                v = c_val[pl.ds(kv * L, L)]
                f = c_flat[pl.ds(kv * L, L)]
                msk = v >= tau2
                pos = off + plsc.cumsum(msk.astype(jnp.int32)) - 1
                plsc.store_scatter(s_v, [pos], v, mask=msk)
                plsc.store_scatter(g_addr, [pos], f, mask=msk)
                return _dg(pos, lane15) + 1
            n2_vec = lax.fori_loop(0, ncv, cc_body, _spl_i(0))
            for m in range(4):
                plsc.store_scatter(s_v, [n2_vec + iota + m * L], _spl_f(NEG))
                plsc.store_scatter(g_addr, [n2_vec + iota + m * L],
                                   _spl_i(IMAX))
            n2 = jnp.max(n2_vec)

            _bitonic_topk(s_v, g_addr, (n2 + 63) // 64, win_val, win_flat)

            for c in range(5):
                for kv in range(4):
                    wf = win_flat[pl.ds(kv * L, L)]
                    beams = wf >> 11
                    g = plsc.load_gather(
                        ids_st, [jnp.broadcast_to(c * K, (L,)) + beams])
                    tmp_st[pl.ds(kv * L, L)] = g
                for kv in range(4):
                    ids_st[pl.ds(c * K + kv * L, L)] = tmp_st[pl.ds(kv * L, L)]
            for kv in range(4):
                wf = win_flat[pl.ds(kv * L, L)]
                score_st[pl.ds(kv * L, L)] = win_val[pl.ds(kv * L, L)]
                t_st[pl.ds(kv * L, L)] = ids_st[pl.ds(kv * L, L)]
                plsc.store_scatter(
                    ids_st,
                    [jnp.broadcast_to((ap + 1) * K + kv * L, (L,)) + iota],
                    wf & (V - 1))

            for kv in range(2):
                aux[pl.ds(kv * L, L)] = _spl_i(0)
            for kv in range(4):
                plsc.store_scatter(aux, [t_st[pl.ds(kv * L, L)]], _spl_i(1))

            def pcomp(kv, off):
                pres = aux[pl.ds(kv * L, L)]
                msk = pres > 0
                pos = off + plsc.cumsum(msk.astype(jnp.int32)) - 1
                tid = iota + kv * L
                plsc.store_scatter(aux, [pos + 32], tid, mask=msk)
                plsc.store_scatter(aux, [tid + 64], pos, mask=msk)
                return _dg(pos, lane15) + 1
            ntv = lax.fori_loop(0, 2, pcomp, _spl_i(0))
            nt = jnp.max(ntv)

            @pl.when(ap < A - 1)
            def _():
                fire_rows(ap + 1, nt)
            return nt

        lax.fori_loop(0, A, ap_body, jnp.int32(T))

        for kv in range(4):
            for c in range(5):
                plsc.store_scatter(
                    outb, [(iota + kv * L) * 5 + c],
                    ids_st[pl.ds(c * K + kv * L, L)])
        pltpu.sync_copy(outb.at[pl.ds(0, 320)], ids_hbm.at[pl.ds(b * 320, 320)])
        pltpu.sync_copy(score_st.at[pl.ds(0, K)], sc_hbm.at[pl.ds(b * K, K)])
        return 0

    lax.fori_loop(0, 2, batch_body, 0)


def kernel(tactic_logits, arg_logits):
    tl_flat = tactic_logits.reshape(-1)
    arg_flat = arg_logits.reshape(-1)
    mesh = plsc.VectorSubcoreMesh(core_axis_name="c", subcore_axis_name="s",
                                  num_cores=2, num_subcores=16)
    f = pl.kernel(
        _body,
        out_type=(
            jax.ShapeDtypeStruct((B * 320,), jnp.int32),
            jax.ShapeDtypeStruct((B * K,), jnp.float32),
        ),
        mesh=mesh,
        compiler_params=pltpu.CompilerParams(needs_layout_passes=False),
        scratch_types=[
            pltpu.VMEM((T * V,), jnp.float32),
            pltpu.VMEM((T * 64,), jnp.float32),
            pltpu.VMEM((K * 64,), jnp.float32),
            pltpu.VMEM((CAP + 128,), jnp.float32),
            pltpu.VMEM((CAP + 128,), jnp.int32),
            pltpu.VMEM((CAP + 128,), jnp.int32),
            pltpu.VMEM((CAP + 128,), jnp.float32),
            pltpu.VMEM((CAP + 128,), jnp.float32),
            pltpu.VMEM((CAP + 128,), jnp.int32),
            pltpu.VMEM((128,), jnp.float32),
            pltpu.VMEM((128,), jnp.int32),
            pltpu.VMEM((128,), jnp.float32),
            pltpu.VMEM((128,), jnp.int32),
            pltpu.VMEM((5 * 128,), jnp.int32),
            pltpu.VMEM((128,), jnp.int32),
            pltpu.VMEM((128,), jnp.float32),
            pltpu.VMEM((384,), jnp.int32),
            pltpu.VMEM((128,), jnp.int32),
            pltpu.SemaphoreType.DMA,
        ],
    )
    ids_f, sc_f = f(tl_flat, arg_flat)
    return ids_f.reshape(B, K, 5), sc_f.reshape(B, K)


# speedup vs baseline: 1.5209x; 1.3194x over previous
import jax
import jax.numpy as jnp
from jax import lax
from jax.experimental import pallas as pl
from jax.experimental.pallas import tpu as pltpu
from jax.experimental.pallas import tpu_sc as plsc

B, T, A, V, K, L = 64, 32, 4, 2048, 64, 16
NEG = float("-inf")
IMAX = 2147483647
CAP = 4096


def _iota():
    return lax.iota(jnp.int32, L)


def _spl_i(x):
    return jnp.broadcast_to(jnp.int32(x), (L,))


def _spl_f(x):
    return jnp.broadcast_to(jnp.float32(x), (L,))


def _dg(x, idx):
    return x.at[idx].get(mode="promise_in_bounds")


def _lexgt(v1, f1, v2, f2):
    return (v1 > v2) | ((v1 == v2) & (f1 < f2))


def _insert4(R, x):
    R0, R1, R2, R3 = R
    m0, m1, m2, m3 = x > R0, x > R1, x > R2, x > R3
    n0 = jnp.where(m0, x, R0)
    n1 = jnp.where(m0, R0, jnp.where(m1, x, R1))
    n2 = jnp.where(m1, R1, jnp.where(m2, x, R2))
    n3 = jnp.where(m2, R2, jnp.where(m3, x, R3))
    return (n0, n1, n2, n3)


def _ce_intra(v, f, j, k, iota):
    isfirst = (iota & j) == 0
    dirv = (iota & k) == 0 if k < L else None
    nv, nf = [], []
    for r in range(4):
        pv, pf = _dg(v[r], iota ^ j), _dg(f[r], iota ^ j)
        takep = _lexgt(pv, pf, v[r], f[r])
        bigv = jnp.where(takep, pv, v[r])
        bigf = jnp.where(takep, pf, f[r])
        smv = jnp.where(takep, v[r], pv)
        smf = jnp.where(takep, f[r], pf)
        if dirv is None:
            want = isfirst if ((r * L) & k) == 0 else ~isfirst
        else:
            want = isfirst == dirv
        nv.append(jnp.where(want, bigv, smv))
        nf.append(jnp.where(want, bigf, smf))
    return nv, nf


def _ce_inter(v, f, j, k):
    step = j // L
    nv, nf = list(v), list(f)
    for ra in range(4):
        rb = ra + step
        if ra & step or rb > 3:
            continue
        takeb = _lexgt(v[rb], f[rb], v[ra], f[ra])
        bigv = jnp.where(takeb, v[rb], v[ra])
        bigf = jnp.where(takeb, f[rb], f[ra])
        smv = jnp.where(takeb, v[ra], v[rb])
        smf = jnp.where(takeb, f[ra], f[rb])
        if ((ra * L) & k) == 0:
            nv[ra], nf[ra], nv[rb], nf[rb] = bigv, bigf, smv, smf
        else:
            nv[ra], nf[ra], nv[rb], nf[rb] = smv, smf, bigv, bigf
    return nv, nf


def _sort64_desc(v, f, iota):
    for k in (2, 4, 8, 16, 32, 64):
        j = k // 2
        while j >= 1:
            if j >= L:
                v, f = _ce_inter(v, f, j, k)
            else:
                v, f = _ce_intra(v, f, j, k, iota)
            j //= 2
    return v, f


def _merge_desc(Rv, Rf, Bv, Bf, iota):
    rev = L - 1 - iota
    v, f = [], []
    for r in range(4):
        brv, brf = _dg(Bv[3 - r], rev), _dg(Bf[3 - r], rev)
        take = _lexgt(brv, brf, Rv[r], Rf[r])
        v.append(jnp.where(take, brv, Rv[r]))
        f.append(jnp.where(take, brf, Rf[r]))
    for j in (32, 16, 8, 4, 2, 1):
        if j >= L:
            v, f = _ce_inter(v, f, j, 64)
        else:
            v, f = _ce_intra(v, f, j, 64, iota)
    return v, f


def _bitonic_topk(cv, cf, nb, wv_ref, wf_ref):
    iota = _iota()

    def blk_body(blk, car):
        Rv, Rf = list(car[0:4]), list(car[4:8])
        base = blk * 64
        Bv = [cv[pl.ds(base + r * L, L)] for r in range(4)]
        Bf = [cf[pl.ds(base + r * L, L)] for r in range(4)]
        Bv, Bf = _sort64_desc(Bv, Bf, iota)
        Rv, Rf = _merge_desc(Rv, Rf, Bv, Bf, iota)
        return (*Rv, *Rf)

    car = lax.fori_loop(
        0, nb, blk_body,
        tuple([_spl_f(NEG)] * 4 + [_spl_i(IMAX)] * 4))
    for r in range(4):
        wv_ref[pl.ds(r * L, L)] = car[r]
        wf_ref[pl.ds(r * L, L)] = car[4 + r]


def _body(tl_hbm, arg_hbm, ids_hbm, sc_hbm,
          rows_v, gmax_v, bounds_v, s_v, g_addr, g_flat, g_sc,
          c_val, c_flat, win_val, win_flat,
          score_st, t_st, ids_st, tmp_st, tl_v, outb, aux, sem):
    iota = _iota()
    cid = lax.axis_index("c")
    sid = lax.axis_index("s")
    wid = sid * 2 + cid

    def batch_body(bi, _):
        b = wid * 2 + bi

        pltpu.sync_copy(tl_hbm.at[pl.ds(b * T, T)], tl_v.at[pl.ds(0, T)])
        for kv in range(2):
            c_val[pl.ds(kv * L, L)] = tl_v[pl.ds(kv * L, L)]
            c_flat[pl.ds(kv * L, L)] = iota + kv * L
        for kv in range(2, 4):
            c_val[pl.ds(kv * L, L)] = _spl_f(NEG)
            c_flat[pl.ds(kv * L, L)] = _spl_i(IMAX)

        for kv in range(2):
            aux[pl.ds(32 + kv * L, L)] = iota + kv * L
            aux[pl.ds(64 + kv * L, L)] = iota + kv * L

        def fire_rows(ap, nt):
            def fire(i, _):
                tid = jnp.max(plsc.load_gather(
                    aux, [jnp.broadcast_to(32 + i, (L,))]))
                pltpu.async_copy(arg_hbm.at[b, tid, ap],
                                 rows_v.at[pl.ds(i * V, V)], sem)
                return 0
            lax.fori_loop(0, nt, fire, 0)

        fire_rows(0, jnp.int32(T))
        _bitonic_topk(c_val, c_flat, 1, win_val, win_flat)
        for kv in range(2):
            w = win_flat[pl.ds(kv * L, L)]
            score_st[pl.ds(kv * L, L)] = win_val[pl.ds(kv * L, L)]
            t_st[pl.ds(kv * L, L)] = w
            ids_st[pl.ds(kv * L, L)] = w
        for kv in range(2, 4):
            score_st[pl.ds(kv * L, L)] = _spl_f(NEG)
            t_st[pl.ds(kv * L, L)] = _spl_i(0)
            ids_st[pl.ds(kv * L, L)] = _spl_i(0)

        def ap_body(ap, nt_prev):
            def dr(i, _):
                pltpu.make_async_copy(arg_hbm.at[0, 0, 0],
                                      rows_v.at[pl.ds(0, V)], sem).wait()
                return 0
            lax.fori_loop(0, nt_prev, dr, 0)

            def gmax_row(t, _):
                tv = t * V
                accs = [_spl_f(NEG) for _ in range(4)]
                for j in range(32):
                    for s in range(4):
                        x = rows_v[pl.ds(tv + (s * L + 64 * j), L)]
                        accs[s] = jnp.maximum(accs[s], x)
                for s in range(4):
                    gmax_v[pl.ds(t * 64 + s * L, L)] = accs[s]
                return 0
            lax.fori_loop(0, nt_prev, gmax_row, 0)

            def bv_body(i, R):
                t_spl = plsc.load_gather(t_st, [jnp.broadcast_to(i, (L,))])
                s_spl = plsc.load_gather(score_st,
                                         [jnp.broadcast_to(i, (L,))])
                sl_spl = plsc.load_gather(aux, [t_spl + 64])
                gb = sl_spl * 64 + iota
                bv = _spl_f(NEG)
                for s in range(4):
                    bnd = plsc.load_gather(gmax_v, [gb + s * L]) + s_spl
                    bounds_v[pl.ds(i * 64 + s * L, L)] = bnd
                    bv = jnp.maximum(bv, bnd)
                return _insert4(R, bv)
            R = lax.fori_loop(0, K, bv_body,
                              (_spl_f(NEG),) * 4, unroll=2)
            tau0 = jnp.min(R[3])

            lane15 = _spl_i(15)

            def sc_scan(kv, off):
                x = bounds_v[pl.ds(kv * L, L)]
                msk = x >= tau0
                pos = off + plsc.cumsum(msk.astype(jnp.int32)) - 1
                plsc.store_scatter(s_v, [pos], x, mask=msk)
                return _dg(pos, lane15) + 1
            n_s_vec = lax.fori_loop(0, K * 4, sc_scan, _spl_i(0), unroll=4)
            plsc.store_scatter(s_v, [n_s_vec + iota], _spl_f(NEG))
            n_s = jnp.max(n_s_vec)

            def s4_body(kv, R):
                return _insert4(R, s_v[pl.ds(kv * L, L)])
            R = lax.fori_loop(0, (n_s + L - 1) // L, s4_body,
                              (_spl_f(NEG),) * 4)
            tau1 = jnp.min(R[3])

            def en_body(i, ng):
                t_spl = plsc.load_gather(t_st, [jnp.broadcast_to(i, (L,))])
                s_spl = plsc.load_gather(score_st,
                                         [jnp.broadcast_to(i, (L,))])
                sl_spl = plsc.load_gather(aux, [t_spl + 64])
                for s in range(4):
                    x = bounds_v[pl.ds(i * 64 + s * L, L)]
                    msk = x >= tau1
                    pos = ng + plsc.cumsum(msk.astype(jnp.int32)) - 1
                    addr = sl_spl * V + iota + s * L
                    flat = jnp.broadcast_to(i * V + s * L, (L,)) + iota
                    plsc.store_scatter(g_addr, [pos], addr, mask=msk)
                    plsc.store_scatter(g_flat, [pos], flat, mask=msk)
                    plsc.store_scatter(g_sc, [pos], s_spl, mask=msk)
                    ng = _dg(pos, lane15) + 1
                return ng
            ng_vec = lax.fori_loop(0, K, en_body, _spl_i(0), unroll=2)
            plsc.store_scatter(g_addr, [ng_vec + iota], _spl_i(0))
            plsc.store_scatter(g_flat, [ng_vec + iota], _spl_i(0))
            plsc.store_scatter(g_sc, [ng_vec + iota], _spl_f(NEG))
            ng = jnp.max(ng_vec)

            def cd_body(gv, nc):
                ab = g_addr[pl.ds(gv * L, L)]
                fb = g_flat[pl.ds(gv * L, L)]
                sc = g_sc[pl.ds(gv * L, L)]
                for j in range(32):
                    val = plsc.load_gather(rows_v, [ab + 64 * j]) + sc
                    msk = val >= tau1
                    pos = nc + plsc.cumsum(msk.astype(jnp.int32)) - 1
                    msk2 = msk & (pos < CAP)
                    plsc.store_scatter(c_val, [pos], val, mask=msk2)
                    plsc.store_scatter(c_flat, [pos], fb + 64 * j, mask=msk2)
                    nc = _dg(pos, lane15) + 1
                return nc
            nc_vec = lax.fori_loop(0, (ng + L - 1) // L, cd_body, _spl_i(0))
            nc_vec = jnp.minimum(nc_vec, jnp.int32(CAP))
            plsc.store_scatter(c_val, [nc_vec + iota], _spl_f(NEG))
            plsc.store_scatter(c_flat, [nc_vec + iota], _spl_i(IMAX))
            nc = jnp.max(nc_vec)

            ncv = (nc + L - 1) // L

            def c4_body(kv, R):
                return _insert4(R, c_val[pl.ds(kv * L, L)])
            R = lax.fori_loop(0, ncv, c4_body, (_spl_f(NEG),) * 4)
            tau2 = jnp.min(R[3])

            def cc_body(kv, off):
                v = c_val[pl.ds(kv * L, L)]
                f = c_flat[pl.ds(kv * L, L)]
                msk = v >= tau2
                pos = off + plsc.cumsum(msk.astype(jnp.int32)) - 1
                plsc.store_scatter(s_v, [pos], v, mask=msk)
                plsc.store_scatter(g_addr, [pos], f, mask=msk)
                return _dg(pos, lane15) + 1
            n2_vec = lax.fori_loop(0, ncv, cc_body, _spl_i(0))
            for m in range(4):
                plsc.store_scatter(s_v, [n2_vec + iota + m * L], _spl_f(NEG))
                plsc.store_scatter(g_addr, [n2_vec + iota + m * L],
                                   _spl_i(IMAX))
            n2 = jnp.max(n2_vec)

            _bitonic_topk(s_v, g_addr, (n2 + 63) // 64, win_val, win_flat)

            for c in range(5):
                for kv in range(4):
                    wf = win_flat[pl.ds(kv * L, L)]
                    beams = wf >> 11
                    g = plsc.load_gather(
                        ids_st, [jnp.broadcast_to(c * K, (L,)) + beams])
                    tmp_st[pl.ds(kv * L, L)] = g
                for kv in range(4):
                    ids_st[pl.ds(c * K + kv * L, L)] = tmp_st[pl.ds(kv * L, L)]
            for kv in range(4):
                wf = win_flat[pl.ds(kv * L, L)]
                score_st[pl.ds(kv * L, L)] = win_val[pl.ds(kv * L, L)]
                t_st[pl.ds(kv * L, L)] = ids_st[pl.ds(kv * L, L)]
                plsc.store_scatter(
                    ids_st,
                    [jnp.broadcast_to((ap + 1) * K + kv * L, (L,)) + iota],
                    wf & (V - 1))

            for kv in range(2):
                aux[pl.ds(kv * L, L)] = _spl_i(0)
            for kv in range(4):
                plsc.store_scatter(aux, [t_st[pl.ds(kv * L, L)]], _spl_i(1))

            def pcomp(kv, off):
                pres = aux[pl.ds(kv * L, L)]
                msk = pres > 0
                pos = off + plsc.cumsum(msk.astype(jnp.int32)) - 1
                tid = iota + kv * L
                plsc.store_scatter(aux, [pos + 32], tid, mask=msk)
                plsc.store_scatter(aux, [tid + 64], pos, mask=msk)
                return _dg(pos, lane15) + 1
            ntv = lax.fori_loop(0, 2, pcomp, _spl_i(0))
            nt = jnp.max(ntv)

            @pl.when(ap < A - 1)
            def _():
                fire_rows(ap + 1, nt)
            return nt

        lax.fori_loop(0, A, ap_body, jnp.int32(T))

        for kv in range(4):
            for c in range(5):
                plsc.store_scatter(
                    outb, [(iota + kv * L) * 5 + c],
                    ids_st[pl.ds(c * K + kv * L, L)])
        pltpu.sync_copy(outb.at[pl.ds(0, 320)], ids_hbm.at[pl.ds(b * 320, 320)])
        pltpu.sync_copy(score_st.at[pl.ds(0, K)], sc_hbm.at[pl.ds(b * K, K)])
        return 0

    lax.fori_loop(0, 2, batch_body, 0)


def kernel(tactic_logits, arg_logits):
    tl_flat = tactic_logits.reshape(-1)
    arg_flat = arg_logits
    mesh = plsc.VectorSubcoreMesh(core_axis_name="c", subcore_axis_name="s",
                                  num_cores=2, num_subcores=16)
    f = pl.kernel(
        _body,
        out_type=(
            jax.ShapeDtypeStruct((B * 320,), jnp.int32),
            jax.ShapeDtypeStruct((B * K,), jnp.float32),
        ),
        mesh=mesh,
        compiler_params=pltpu.CompilerParams(needs_layout_passes=False),
        scratch_types=[
            pltpu.VMEM((T * V,), jnp.float32),
            pltpu.VMEM((T * 64,), jnp.float32),
            pltpu.VMEM((K * 64,), jnp.float32),
            pltpu.VMEM((CAP + 128,), jnp.float32),
            pltpu.VMEM((CAP + 128,), jnp.int32),
            pltpu.VMEM((CAP + 128,), jnp.int32),
            pltpu.VMEM((CAP + 128,), jnp.float32),
            pltpu.VMEM((CAP + 128,), jnp.float32),
            pltpu.VMEM((CAP + 128,), jnp.int32),
            pltpu.VMEM((128,), jnp.float32),
            pltpu.VMEM((128,), jnp.int32),
            pltpu.VMEM((128,), jnp.float32),
            pltpu.VMEM((128,), jnp.int32),
            pltpu.VMEM((5 * 128,), jnp.int32),
            pltpu.VMEM((128,), jnp.int32),
            pltpu.VMEM((128,), jnp.float32),
            pltpu.VMEM((384,), jnp.int32),
            pltpu.VMEM((128,), jnp.int32),
            pltpu.SemaphoreType.DMA,
        ],
    )
    ids_f, sc_f = f(tl_flat, arg_flat)
    return ids_f.reshape(B, K, 5), sc_f.reshape(B, K)


# speedup vs baseline: 1.5292x; 1.0054x over previous
import jax
import jax.numpy as jnp
from jax import lax
from jax.experimental import pallas as pl
from jax.experimental.pallas import tpu as pltpu
from jax.experimental.pallas import tpu_sc as plsc

B, T, A, V, K, L = 64, 32, 4, 2048, 64, 16
NEG = float("-inf")
IMAX = 2147483647
CAP = 4096


def _iota():
    return lax.iota(jnp.int32, L)


def _spl_i(x):
    return jnp.broadcast_to(jnp.int32(x), (L,))


def _spl_f(x):
    return jnp.broadcast_to(jnp.float32(x), (L,))


def _dg(x, idx):
    return x.at[idx].get(mode="promise_in_bounds")


def _lexgt(v1, f1, v2, f2):
    return (v1 > v2) | ((v1 == v2) & (f1 < f2))


def _insert4(R, x):
    R0, R1, R2, R3 = R
    m0, m1, m2, m3 = x > R0, x > R1, x > R2, x > R3
    n0 = jnp.where(m0, x, R0)
    n1 = jnp.where(m0, R0, jnp.where(m1, x, R1))
    n2 = jnp.where(m1, R1, jnp.where(m2, x, R2))
    n3 = jnp.where(m2, R2, jnp.where(m3, x, R3))
    return (n0, n1, n2, n3)


def _ce_intra(v, f, j, k, iota):
    isfirst = (iota & j) == 0
    dirv = (iota & k) == 0 if k < L else None
    nv, nf = [], []
    for r in range(4):
        pv, pf = _dg(v[r], iota ^ j), _dg(f[r], iota ^ j)
        takep = _lexgt(pv, pf, v[r], f[r])
        bigv = jnp.where(takep, pv, v[r])
        bigf = jnp.where(takep, pf, f[r])
        smv = jnp.where(takep, v[r], pv)
        smf = jnp.where(takep, f[r], pf)
        if dirv is None:
            want = isfirst if ((r * L) & k) == 0 else ~isfirst
        else:
            want = isfirst == dirv
        nv.append(jnp.where(want, bigv, smv))
        nf.append(jnp.where(want, bigf, smf))
    return nv, nf


def _ce_inter(v, f, j, k):
    step = j // L
    nv, nf = list(v), list(f)
    for ra in range(4):
        rb = ra + step
        if ra & step or rb > 3:
            continue
        takeb = _lexgt(v[rb], f[rb], v[ra], f[ra])
        bigv = jnp.where(takeb, v[rb], v[ra])
        bigf = jnp.where(takeb, f[rb], f[ra])
        smv = jnp.where(takeb, v[ra], v[rb])
        smf = jnp.where(takeb, f[ra], f[rb])
        if ((ra * L) & k) == 0:
            nv[ra], nf[ra], nv[rb], nf[rb] = bigv, bigf, smv, smf
        else:
            nv[ra], nf[ra], nv[rb], nf[rb] = smv, smf, bigv, bigf
    return nv, nf


def _sort64_desc(v, f, iota):
    for k in (2, 4, 8, 16, 32, 64):
        j = k // 2
        while j >= 1:
            if j >= L:
                v, f = _ce_inter(v, f, j, k)
            else:
                v, f = _ce_intra(v, f, j, k, iota)
            j //= 2
    return v, f


def _merge_desc(Rv, Rf, Bv, Bf, iota):
    rev = L - 1 - iota
    v, f = [], []
    for r in range(4):
        brv, brf = _dg(Bv[3 - r], rev), _dg(Bf[3 - r], rev)
        take = _lexgt(brv, brf, Rv[r], Rf[r])
        v.append(jnp.where(take, brv, Rv[r]))
        f.append(jnp.where(take, brf, Rf[r]))
    for j in (32, 16, 8, 4, 2, 1):
        if j >= L:
            v, f = _ce_inter(v, f, j, 64)
        else:
            v, f = _ce_intra(v, f, j, 64, iota)
    return v, f


def _bitonic_topk(cv, cf, nb, wv_ref, wf_ref):
    iota = _iota()

    def blk_body(blk, car):
        Rv, Rf = list(car[0:4]), list(car[4:8])
        base = blk * 64
        Bv = [cv[pl.ds(base + r * L, L)] for r in range(4)]
        Bf = [cf[pl.ds(base + r * L, L)] for r in range(4)]
        Bv, Bf = _sort64_desc(Bv, Bf, iota)
        Rv, Rf = _merge_desc(Rv, Rf, Bv, Bf, iota)
        return (*Rv, *Rf)

    car = lax.fori_loop(
        0, nb, blk_body,
        tuple([_spl_f(NEG)] * 4 + [_spl_i(IMAX)] * 4))
    for r in range(4):
        wv_ref[pl.ds(r * L, L)] = car[r]
        wf_ref[pl.ds(r * L, L)] = car[4 + r]


def _body(tl_hbm, arg_hbm, ids_hbm, sc_hbm,
          rows_v, gmax_v, bounds_v, s_v, g_addr, g_flat, g_sc,
          c_val, c_flat, win_val, win_flat,
          score_st, t_st, ids_st, tmp_st, tl_v, outb, aux, sem):
    iota = _iota()
    cid = lax.axis_index("c")
    sid = lax.axis_index("s")
    wid = sid * 2 + cid

    def batch_body(bi, _):
        b = wid * 2 + bi

        pltpu.sync_copy(tl_hbm.at[pl.ds(b * T, T)], tl_v.at[pl.ds(0, T)])
        for kv in range(2):
            c_val[pl.ds(kv * L, L)] = tl_v[pl.ds(kv * L, L)]
            c_flat[pl.ds(kv * L, L)] = iota + kv * L
        for kv in range(2, 4):
            c_val[pl.ds(kv * L, L)] = _spl_f(NEG)
            c_flat[pl.ds(kv * L, L)] = _spl_i(IMAX)

        for kv in range(2):
            aux[pl.ds(32 + kv * L, L)] = iota + kv * L
            aux[pl.ds(64 + kv * L, L)] = iota + kv * L

        def fire_rows(ap, nt):
            def fire(i, _):
                tid = jnp.max(plsc.load_gather(
                    aux, [jnp.broadcast_to(32 + i, (L,))]))
                pltpu.async_copy(arg_hbm.at[b, tid, ap],
                                 rows_v.at[pl.ds(i * V, V)], sem)
                return 0
            lax.fori_loop(0, nt, fire, 0)

        fire_rows(0, jnp.int32(T))
        _bitonic_topk(c_val, c_flat, 1, win_val, win_flat)
        for kv in range(2):
            w = win_flat[pl.ds(kv * L, L)]
            score_st[pl.ds(kv * L, L)] = win_val[pl.ds(kv * L, L)]
            t_st[pl.ds(kv * L, L)] = w
            ids_st[pl.ds(kv * L, L)] = w
        for kv in range(2, 4):
            score_st[pl.ds(kv * L, L)] = _spl_f(NEG)
            t_st[pl.ds(kv * L, L)] = _spl_i(0)
            ids_st[pl.ds(kv * L, L)] = _spl_i(0)

        def ap_body(ap, nt_prev):
            def dr(i, _):
                pltpu.make_async_copy(arg_hbm.at[0, 0, 0],
                                      rows_v.at[pl.ds(0, V)], sem).wait()
                return 0
            lax.fori_loop(0, nt_prev, dr, 0)

            def gmax_row(t, _):
                tv = t * V
                accs = [_spl_f(NEG) for _ in range(4)]
                for j in range(32):
                    for s in range(4):
                        x = rows_v[pl.ds(tv + (s * L + 64 * j), L)]
                        accs[s] = jnp.maximum(accs[s], x)
                for s in range(4):
                    gmax_v[pl.ds(t * 64 + s * L, L)] = accs[s]
                return 0
            lax.fori_loop(0, nt_prev, gmax_row, 0)

            def bv_body(i, R):
                t_spl = plsc.load_gather(t_st, [jnp.broadcast_to(i, (L,))])
                s_spl = plsc.load_gather(score_st,
                                         [jnp.broadcast_to(i, (L,))])
                sl_spl = plsc.load_gather(aux, [t_spl + 64])
                gb = sl_spl * 64 + iota
                bv = _spl_f(NEG)
                for s in range(4):
                    bnd = plsc.load_gather(gmax_v, [gb + s * L]) + s_spl
                    bounds_v[pl.ds(i * 64 + s * L, L)] = bnd
                    bv = jnp.maximum(bv, bnd)
                return _insert4(R, bv)
            R = lax.fori_loop(0, K, bv_body,
                              (_spl_f(NEG),) * 4, unroll=4)
            tau0 = jnp.min(R[3])

            lane15 = _spl_i(15)

            def sc_scan(kv, off):
                x = bounds_v[pl.ds(kv * L, L)]
                msk = x >= tau0
                pos = off + plsc.cumsum(msk.astype(jnp.int32)) - 1
                plsc.store_scatter(s_v, [pos], x, mask=msk)
                return _dg(pos, lane15) + 1
            n_s_vec = lax.fori_loop(0, K * 4, sc_scan, _spl_i(0), unroll=8)
            plsc.store_scatter(s_v, [n_s_vec + iota], _spl_f(NEG))
            n_s = jnp.max(n_s_vec)

            def s4_body(kv, R):
                return _insert4(R, s_v[pl.ds(kv * L, L)])
            R = lax.fori_loop(0, (n_s + L - 1) // L, s4_body,
                              (_spl_f(NEG),) * 4)
            tau1 = jnp.min(R[3])

            def en_body(i, ng):
                t_spl = plsc.load_gather(t_st, [jnp.broadcast_to(i, (L,))])
                s_spl = plsc.load_gather(score_st,
                                         [jnp.broadcast_to(i, (L,))])
                sl_spl = plsc.load_gather(aux, [t_spl + 64])
                for s in range(4):
                    x = bounds_v[pl.ds(i * 64 + s * L, L)]
                    msk = x >= tau1
                    pos = ng + plsc.cumsum(msk.astype(jnp.int32)) - 1
                    addr = sl_spl * V + iota + s * L
                    flat = jnp.broadcast_to(i * V + s * L, (L,)) + iota
                    plsc.store_scatter(g_addr, [pos], addr, mask=msk)
                    plsc.store_scatter(g_flat, [pos], flat, mask=msk)
                    plsc.store_scatter(g_sc, [pos], s_spl, mask=msk)
                    ng = _dg(pos, lane15) + 1
                return ng
            ng_vec = lax.fori_loop(0, K, en_body, _spl_i(0), unroll=4)
            plsc.store_scatter(g_addr, [ng_vec + iota], _spl_i(0))
            plsc.store_scatter(g_flat, [ng_vec + iota], _spl_i(0))
            plsc.store_scatter(g_sc, [ng_vec + iota], _spl_f(NEG))
            ng = jnp.max(ng_vec)

            def cd_body(gv, nc):
                ab = g_addr[pl.ds(gv * L, L)]
                fb = g_flat[pl.ds(gv * L, L)]
                sc = g_sc[pl.ds(gv * L, L)]
                for j in range(32):
                    val = plsc.load_gather(rows_v, [ab + 64 * j]) + sc
                    msk = val >= tau1
                    pos = nc + plsc.cumsum(msk.astype(jnp.int32)) - 1
                    msk2 = msk & (pos < CAP)
                    plsc.store_scatter(c_val, [pos], val, mask=msk2)
                    plsc.store_scatter(c_flat, [pos], fb + 64 * j, mask=msk2)
                    nc = _dg(pos, lane15) + 1
                return nc
            nc_vec = lax.fori_loop(0, (ng + L - 1) // L, cd_body, _spl_i(0))
            nc_vec = jnp.minimum(nc_vec, jnp.int32(CAP))
            plsc.store_scatter(c_val, [nc_vec + iota], _spl_f(NEG))
            plsc.store_scatter(c_flat, [nc_vec + iota], _spl_i(IMAX))
            nc = jnp.max(nc_vec)

            ncv = (nc + L - 1) // L

            def c4_body(kv, R):
                return _insert4(R, c_val[pl.ds(kv * L, L)])
            R = lax.fori_loop(0, ncv, c4_body, (_spl_f(NEG),) * 4)
            tau2 = jnp.min(R[3])

            def cc_body(kv, off):
                v = c_val[pl.ds(kv * L, L)]
                f = c_flat[pl.ds(kv * L, L)]
                msk = v >= tau2
                pos = off + plsc.cumsum(msk.astype(jnp.int32)) - 1
                plsc.store_scatter(s_v, [pos], v, mask=msk)
                plsc.store_scatter(g_addr, [pos], f, mask=msk)
                return _dg(pos, lane15) + 1
            n2_vec = lax.fori_loop(0, ncv, cc_body, _spl_i(0))
            for m in range(4):
                plsc.store_scatter(s_v, [n2_vec + iota + m * L], _spl_f(NEG))
                plsc.store_scatter(g_addr, [n2_vec + iota + m * L],
                                   _spl_i(IMAX))
            n2 = jnp.max(n2_vec)

            _bitonic_topk(s_v, g_addr, (n2 + 63) // 64, win_val, win_flat)

            for c in range(5):
                for kv in range(4):
                    wf = win_flat[pl.ds(kv * L, L)]
                    beams = wf >> 11
                    g = plsc.load_gather(
                        ids_st, [jnp.broadcast_to(c * K, (L,)) + beams])
                    tmp_st[pl.ds(kv * L, L)] = g
                for kv in range(4):
                    ids_st[pl.ds(c * K + kv * L, L)] = tmp_st[pl.ds(kv * L, L)]
            for kv in range(4):
                wf = win_flat[pl.ds(kv * L, L)]
                score_st[pl.ds(kv * L, L)] = win_val[pl.ds(kv * L, L)]
                t_st[pl.ds(kv * L, L)] = ids_st[pl.ds(kv * L, L)]
                plsc.store_scatter(
                    ids_st,
                    [jnp.broadcast_to((ap + 1) * K + kv * L, (L,)) + iota],
                    wf & (V - 1))

            for kv in range(2):
                aux[pl.ds(kv * L, L)] = _spl_i(0)
            for kv in range(4):
                plsc.store_scatter(aux, [t_st[pl.ds(kv * L, L)]], _spl_i(1))

            def pcomp(kv, off):
                pres = aux[pl.ds(kv * L, L)]
                msk = pres > 0
                pos = off + plsc.cumsum(msk.astype(jnp.int32)) - 1
                tid = iota + kv * L
                plsc.store_scatter(aux, [pos + 32], tid, mask=msk)
                plsc.store_scatter(aux, [tid + 64], pos, mask=msk)
                return _dg(pos, lane15) + 1
            ntv = lax.fori_loop(0, 2, pcomp, _spl_i(0))
            nt = jnp.max(ntv)

            @pl.when(ap < A - 1)
            def _():
                fire_rows(ap + 1, nt)
            return nt

        lax.fori_loop(0, A, ap_body, jnp.int32(T))

        for kv in range(4):
            for c in range(5):
                plsc.store_scatter(
                    outb, [(iota + kv * L) * 5 + c],
                    ids_st[pl.ds(c * K + kv * L, L)])
        pltpu.sync_copy(outb.at[pl.ds(0, 320)], ids_hbm.at[pl.ds(b * 320, 320)])
        pltpu.sync_copy(score_st.at[pl.ds(0, K)], sc_hbm.at[pl.ds(b * K, K)])
        return 0

    lax.fori_loop(0, 2, batch_body, 0)


def kernel(tactic_logits, arg_logits):
    tl_flat = tactic_logits.reshape(-1)
    arg_flat = arg_logits
    mesh = plsc.VectorSubcoreMesh(core_axis_name="c", subcore_axis_name="s",
                                  num_cores=2, num_subcores=16)
    f = pl.kernel(
        _body,
        out_type=(
            jax.ShapeDtypeStruct((B * 320,), jnp.int32),
            jax.ShapeDtypeStruct((B * K,), jnp.float32),
        ),
        mesh=mesh,
        compiler_params=pltpu.CompilerParams(needs_layout_passes=False),
        scratch_types=[
            pltpu.VMEM((T * V,), jnp.float32),
            pltpu.VMEM((T * 64,), jnp.float32),
            pltpu.VMEM((K * 64,), jnp.float32),
            pltpu.VMEM((CAP + 128,), jnp.float32),
            pltpu.VMEM((CAP + 128,), jnp.int32),
            pltpu.VMEM((CAP + 128,), jnp.int32),
            pltpu.VMEM((CAP + 128,), jnp.float32),
            pltpu.VMEM((CAP + 128,), jnp.float32),
            pltpu.VMEM((CAP + 128,), jnp.int32),
            pltpu.VMEM((128,), jnp.float32),
            pltpu.VMEM((128,), jnp.int32),
            pltpu.VMEM((128,), jnp.float32),
            pltpu.VMEM((128,), jnp.int32),
            pltpu.VMEM((5 * 128,), jnp.int32),
            pltpu.VMEM((128,), jnp.int32),
            pltpu.VMEM((128,), jnp.float32),
            pltpu.VMEM((384,), jnp.int32),
            pltpu.VMEM((128,), jnp.int32),
            pltpu.SemaphoreType.DMA,
        ],
    )
    ids_f, sc_f = f(tl_flat, arg_flat)
    return ids_f.reshape(B, K, 5), sc_f.reshape(B, K)
